# Initial kernel scaffold; baseline (speedup 1.0000x reference)
import jax
import jax.numpy as jnp
from jax.experimental import pallas as pl


def kernel(tm1_input, weights, source_indices, target_indices, type_ids, steps):
    raise NotImplementedError("write your pallas kernel here")


# speedup vs baseline: 64.8294x; 64.8294x over previous
import functools

import jax
import jax.numpy as jnp
from jax import lax
from jax.experimental import pallas as pl
from jax.experimental.pallas import tpu as pltpu
from jax.experimental.pallas import tpu_sc as plsc

N_NEURONS = 100000
DT = 0.1
DECAY = 1.0 - DT

NC = 2
NS = 16
NW = NC * NS
L = 16

TPT = 3136
NPAD = NW * TPT
TBITS = 13
TMASK = (1 << TBITS) - 1
CHUNK = 2048
UNROLL = 16
SUB = CHUNK // L // UNROLL


def _step(st_hbm, w_hbm, starts_hbm, m_hbm, tm1_hbm, v_hbm, r_hbm,
          v_out, r_out,
          r_full, acc, vsl, msl, tsl, rsl, stb0, wb0, stb1, wb1, stv,
          sem0, sem1):
    wid = lax.axis_index("c") * NS + lax.axis_index("s")
    off0 = pl.multiple_of(wid * TPT, 8)

    rcp = pltpu.make_async_copy(r_hbm, r_full, sem1)
    rcp.start()
    pltpu.sync_copy(starts_hbm, stv)
    pltpu.sync_copy(v_hbm.at[pl.ds(off0, TPT)], vsl)
    pltpu.sync_copy(m_hbm.at[pl.ds(off0, TPT)], msl)
    pltpu.sync_copy(tm1_hbm.at[pl.ds(off0, TPT)], tsl)

    start = jnp.max(plsc.load_gather(stv, [jnp.full((L,), wid, jnp.int32)]))
    end = jnp.max(plsc.load_gather(stv, [jnp.full((L,), wid + 1, jnp.int32)]))
    base = jnp.bitwise_and(start, jnp.int32(-8))
    nch = (end - base + (CHUNK - 1)) // CHUNK

    zeros = jnp.zeros((L,), jnp.float32)

    def _zero(i, carry):
        acc[pl.ds(i * L, L)] = zeros
        return carry

    lax.fori_loop(0, TPT // L, _zero, 0)

    iota = lax.broadcasted_iota(jnp.int32, (L,), 0)
    rcp.wait()

    def _process(stbuf, wbuf, off):
        def _inner(k, c):
            o = k * (UNROLL * L)
            for u in range(UNROLL):
                oo = o + u * L
                st = stbuf[pl.ds(oo, L)]
                sv = lax.shift_right_logical(st, TBITS)
                tv = jnp.bitwise_and(st, jnp.int32(TMASK))
                wv = wbuf[pl.ds(oo, L)]
                pos = iota + (off + oo)
                ok = jnp.logical_and(pos >= start, pos < end)
                wm = jnp.where(ok, wv, 0.0)
                vals = plsc.load_gather(r_full, [sv])
                plsc.addupdate_scatter(acc, [tv], vals * wm)
            return c

        lax.fori_loop(0, SUB, _inner, 0)

    offp = pl.multiple_of(base, 8)
    pltpu.make_async_copy(st_hbm.at[pl.ds(offp, CHUNK)], stb0, sem0).start()
    pltpu.make_async_copy(w_hbm.at[pl.ds(offp, CHUNK)], wb0, sem0).start()

    def _chunk2(p, c):
        offa = pl.multiple_of(base + (2 * p) * CHUNK, 8)
        offb = pl.multiple_of(base + (2 * p + 1) * CHUNK, 8)
        offc = pl.multiple_of(base + (2 * p + 2) * CHUNK, 8)
        pltpu.make_async_copy(st_hbm.at[pl.ds(offa, CHUNK)], stb0, sem0).wait()
        pltpu.make_async_copy(w_hbm.at[pl.ds(offa, CHUNK)], wb0, sem0).wait()
        pltpu.make_async_copy(st_hbm.at[pl.ds(offb, CHUNK)], stb1, sem1).start()
        pltpu.make_async_copy(w_hbm.at[pl.ds(offb, CHUNK)], wb1, sem1).start()
        _process(stb0, wb0, offa)
        pltpu.make_async_copy(st_hbm.at[pl.ds(offb, CHUNK)], stb1, sem1).wait()
        pltpu.make_async_copy(w_hbm.at[pl.ds(offb, CHUNK)], wb1, sem1).wait()
        pltpu.make_async_copy(st_hbm.at[pl.ds(offc, CHUNK)], stb0, sem0).start()
        pltpu.make_async_copy(w_hbm.at[pl.ds(offc, CHUNK)], wb0, sem0).start()
        _process(stb1, wb1, offb)
        return c

    npairs = (nch + 1) // 2
    lax.fori_loop(0, npairs, _chunk2, 0)
    offz = pl.multiple_of(base + 2 * npairs * CHUNK, 8)
    pltpu.make_async_copy(st_hbm.at[pl.ds(offz, CHUNK)], stb0, sem0).wait()
    pltpu.make_async_copy(w_hbm.at[pl.ds(offz, CHUNK)], wb0, sem0).wait()

    def _upd(i, carry):
        ds = pl.ds(i * L, L)
        syn = acc[ds]
        v = vsl[ds]
        m = msl[ds]
        t = tsl[ds]
        vn = v * DECAY + syn * DT
        vn = vn * (1.0 - m) + t * m
        vsl[ds] = vn
        rsl[ds] = jnp.maximum(vn, 0.0)
        return carry

    lax.fori_loop(0, TPT // L, _upd, 0)

    pltpu.sync_copy(vsl, v_out.at[pl.ds(off0, TPT)])
    pltpu.sync_copy(rsl, r_out.at[pl.ds(off0, TPT)])


_step_call = functools.partial(
    pl.kernel,
    out_type=(
        jax.ShapeDtypeStruct((NPAD,), jnp.float32),
        jax.ShapeDtypeStruct((NPAD,), jnp.float32),
    ),
    mesh=plsc.VectorSubcoreMesh(
        core_axis_name="c", subcore_axis_name="s", num_cores=NC,
        num_subcores=NS,
    ),
    compiler_params=pltpu.CompilerParams(needs_layout_passes=False),
    scratch_types=(
        pltpu.VMEM((NPAD,), jnp.float32),
        pltpu.VMEM((TPT,), jnp.float32),
        pltpu.VMEM((TPT,), jnp.float32),
        pltpu.VMEM((TPT,), jnp.float32),
        pltpu.VMEM((TPT,), jnp.float32),
        pltpu.VMEM((TPT,), jnp.float32),
        pltpu.VMEM((CHUNK,), jnp.int32),
        pltpu.VMEM((CHUNK,), jnp.float32),
        pltpu.VMEM((CHUNK,), jnp.int32),
        pltpu.VMEM((CHUNK,), jnp.float32),
        pltpu.VMEM((48,), jnp.int32),
        pltpu.SemaphoreType.DMA,
        pltpu.SemaphoreType.DMA,
    ),
)(_step)


def kernel(tm1_input, weights, source_indices, target_indices, type_ids, steps):
    e = weights.shape[0]
    e_pad = e + 4 * CHUNK

    t_s, s_s, w_s = lax.sort(
        [target_indices, source_indices, weights], num_keys=1)
    st = s_s * (1 << TBITS) + jnp.remainder(t_s, TPT)

    st_e = jnp.zeros((e_pad,), jnp.int32).at[:e].set(st)
    w_e = jnp.zeros((e_pad,), jnp.float32).at[:e].set(w_s)

    bounds = jnp.arange(NW, dtype=jnp.int32) * TPT
    starts = jnp.searchsorted(t_s, bounds).astype(jnp.int32)
    starts = jnp.concatenate(
        [starts, jnp.full((48 - NW,), e, dtype=jnp.int32)])

    tm1_idx = jnp.nonzero(type_ids == 0, size=tm1_input.shape[1])[0]
    m = jnp.zeros((NPAD,), jnp.float32).at[tm1_idx].set(1.0)
    tm1f = jnp.zeros((NPAD,), jnp.float32).at[tm1_idx].set(tm1_input[0])
    v0 = tm1f
    r0 = jnp.maximum(v0, 0.0)

    def body(_, carry):
        v, r = carry
        return _step_call(st_e, w_e, starts, m, tm1f, v, r)

    v_fin, _ = lax.fori_loop(0, steps, body, (v0, r0))
    return v_fin[:N_NEURONS].reshape(1, N_NEURONS)
